# baseline (speedup 1.0000x reference)
import functools

import jax
import jax.numpy as jnp
from jax.experimental import pallas as pl
from jax.experimental.pallas import tpu as pltpu


def _ffn_kernel(x_ref, w1_ref, b1_ref, w2_ref, b2_ref, o_ref, *, nc):
    xb = x_ref[...].astype(jnp.bfloat16)
    dh = w1_ref.shape[1]
    ch = dh // nc
    acc = None
    for c in range(nc):
        sl = slice(c * ch, (c + 1) * ch)
        h = jnp.dot(xb, w1_ref[:, sl], preferred_element_type=jnp.float32)
        h = h + b1_ref[:, sl]
        h = jax.nn.gelu(h, approximate=True)
        y = jnp.dot(h.astype(jnp.bfloat16), w2_ref[sl, :],
                    preferred_element_type=jnp.float32)
        acc = y if acc is None else acc + y
    o_ref[...] = acc + b2_ref[...]


def _row_tile(m, target):
    if m % target == 0:
        return target
    t = (min(m, target) // 8) * 8
    while t >= 8:
        if m % t == 0:
            return t
        t -= 8
    return m


def kernel(x, w1, b1, w2, b2):
    b, n, d = x.shape
    dh = w1.shape[1]
    m = b * n
    x2 = x.reshape(m, d)
    w1b = w1.astype(jnp.bfloat16)
    w2b = w2.astype(jnp.bfloat16)

    tm = _row_tile(m, 2048)
    grid = (m // tm,)
    cost = pl.CostEstimate(
        flops=4 * m * d * dh,
        transcendentals=m * dh,
        bytes_accessed=(m * d * 2) * 4 + (2 * d * dh + d + dh) * 2,
    )
    out = pl.pallas_call(
        functools.partial(_ffn_kernel, nc=1),
        out_shape=jax.ShapeDtypeStruct((m, d), x.dtype),
        grid=grid,
        in_specs=[
            pl.BlockSpec((tm, d), lambda i: (i, 0)),
            pl.BlockSpec((d, dh), lambda i: (0, 0)),
            pl.BlockSpec((1, dh), lambda i: (0, 0)),
            pl.BlockSpec((dh, d), lambda i: (0, 0)),
            pl.BlockSpec((1, d), lambda i: (0, 0)),
        ],
        out_specs=pl.BlockSpec((tm, d), lambda i: (i, 0)),
        compiler_params=pltpu.CompilerParams(
            dimension_semantics=("parallel",),
            vmem_limit_bytes=100 * 1024 * 1024,
        ),
        cost_estimate=cost,
    )(x2, w1b, b1, w2b, b2)
    return out.reshape(b, n, d)


# speedup vs baseline: 1.1700x; 1.1700x over previous
import jax
import jax.numpy as jnp
from jax.experimental import pallas as pl
from jax.experimental.pallas import tpu as pltpu


def _ffn_kernel(x_ref, w1_ref, b1_ref, w2_ref, b2_ref, o_ref,
                w1s_ref, w2s_ref):
    @pl.when(pl.program_id(1) == 0)
    def _():
        w1s_ref[...] = w1_ref[...].astype(jnp.bfloat16)
        w2s_ref[...] = w2_ref[...].astype(jnp.bfloat16)

    xb = x_ref[...].astype(jnp.bfloat16)
    h = jnp.dot(xb, w1s_ref[...], preferred_element_type=jnp.float32)
    h = h + b1_ref[...]
    h = jax.nn.gelu(h, approximate=True)
    y = jnp.dot(h.astype(jnp.bfloat16), w2s_ref[...],
                preferred_element_type=jnp.float32)
    o_ref[...] = y + b2_ref[...]


def _row_tile(m, target):
    if m % target == 0:
        return target
    t = (min(m, target) // 8) * 8
    while t >= 8:
        if m % t == 0:
            return t
        t -= 8
    return m


def kernel(x, w1, b1, w2, b2):
    b, n, d = x.shape
    dh = w1.shape[1]
    m = b * n
    x2 = x.reshape(m, d)

    tm = _row_tile(m, 512)
    nrow = m // tm
    ncore = 2 if nrow % 2 == 0 else 1
    nin = nrow // ncore
    cost = pl.CostEstimate(
        flops=4 * m * d * dh,
        transcendentals=m * dh,
        bytes_accessed=(m * d * 2 + 2 * d * dh + d + dh) * 4,
    )
    out = pl.pallas_call(
        _ffn_kernel,
        out_shape=jax.ShapeDtypeStruct((m, d), x.dtype),
        grid_spec=pltpu.PrefetchScalarGridSpec(
            num_scalar_prefetch=0,
            grid=(ncore, nin),
            in_specs=[
                pl.BlockSpec((tm, d), lambda i, j: (i * nin + j, 0)),
                pl.BlockSpec((d, dh), lambda i, j: (0, 0)),
                pl.BlockSpec((1, dh), lambda i, j: (0, 0)),
                pl.BlockSpec((dh, d), lambda i, j: (0, 0)),
                pl.BlockSpec((1, d), lambda i, j: (0, 0)),
            ],
            out_specs=pl.BlockSpec((tm, d), lambda i, j: (i * nin + j, 0)),
            scratch_shapes=[
                pltpu.VMEM((d, dh), jnp.bfloat16),
                pltpu.VMEM((dh, d), jnp.bfloat16),
            ],
        ),
        compiler_params=pltpu.CompilerParams(
            dimension_semantics=("parallel", "arbitrary"),
            vmem_limit_bytes=100 * 1024 * 1024,
        ),
        cost_estimate=cost,
    )(x2, w1, b1, w2, b2)
    return out.reshape(b, n, d)


# speedup vs baseline: 1.2005x; 1.0261x over previous
import jax
import jax.numpy as jnp
from jax.experimental import pallas as pl
from jax.experimental.pallas import tpu as pltpu


def _ffn_kernel(x_ref, w1_ref, b1_ref, w2_ref, b2_ref, o_ref,
                w1s_ref, w2s_ref):
    @pl.when(pl.program_id(1) == 0)
    def _():
        w1s_ref[...] = w1_ref[...].astype(jnp.bfloat16)
        w2s_ref[...] = w2_ref[...].astype(jnp.bfloat16)

    xb = x_ref[...].astype(jnp.bfloat16)
    h = jnp.dot(xb, w1s_ref[...], preferred_element_type=jnp.float32)
    h = h + b1_ref[...]
    h = jax.nn.gelu(h, approximate=True)
    y = jnp.dot(h.astype(jnp.bfloat16), w2s_ref[...],
                preferred_element_type=jnp.float32)
    o_ref[...] = y + b2_ref[...]


def _row_tile(m, target):
    if m % target == 0:
        return target
    t = (min(m, target) // 8) * 8
    while t >= 8:
        if m % t == 0:
            return t
        t -= 8
    return m


def kernel(x, w1, b1, w2, b2):
    b, n, d = x.shape
    dh = w1.shape[1]
    m = b * n
    x2 = x.reshape(m, d)

    tm = _row_tile(m, 1024)
    nrow = m // tm
    ncore = 2 if nrow % 2 == 0 else 1
    nin = nrow // ncore
    cost = pl.CostEstimate(
        flops=4 * m * d * dh,
        transcendentals=m * dh,
        bytes_accessed=(m * d * 2 + 2 * d * dh + d + dh) * 4,
    )
    out = pl.pallas_call(
        _ffn_kernel,
        out_shape=jax.ShapeDtypeStruct((m, d), x.dtype),
        grid_spec=pltpu.PrefetchScalarGridSpec(
            num_scalar_prefetch=0,
            grid=(ncore, nin),
            in_specs=[
                pl.BlockSpec((tm, d), lambda i, j: (i * nin + j, 0)),
                pl.BlockSpec((d, dh), lambda i, j: (0, 0)),
                pl.BlockSpec((1, dh), lambda i, j: (0, 0)),
                pl.BlockSpec((dh, d), lambda i, j: (0, 0)),
                pl.BlockSpec((1, d), lambda i, j: (0, 0)),
            ],
            out_specs=pl.BlockSpec((tm, d), lambda i, j: (i * nin + j, 0)),
            scratch_shapes=[
                pltpu.VMEM((d, dh), jnp.bfloat16),
                pltpu.VMEM((dh, d), jnp.bfloat16),
            ],
        ),
        compiler_params=pltpu.CompilerParams(
            dimension_semantics=("parallel", "arbitrary"),
            vmem_limit_bytes=100 * 1024 * 1024,
        ),
        cost_estimate=cost,
    )(x2, w1, b1, w2, b2)
    return out.reshape(b, n, d)
